# Initial kernel scaffold; baseline (speedup 1.0000x reference)
import jax
import jax.numpy as jnp
from jax.experimental import pallas as pl


def kernel(x, edge_index, edge_attr, eps, W, b, gamma, beta):
    raise NotImplementedError("write your pallas kernel here")


# speedup vs baseline: 2.9884x; 2.9884x over previous
import functools

import jax
import jax.numpy as jnp
from jax import lax
from jax.experimental import pallas as pl
from jax.experimental.pallas import tpu as pltpu
from jax.experimental.pallas import tpu_sc as plsc

N = 10000
E = 320000
D = 128
L = 3

NC = 2
NS = 16
NP = 10240
WPT = NP // NS
EPSC = E // NC
EPT = EPSC // NS
B = 80
NCH = EPT // B


def _sc_agg(h, edge_attr, src, dst):
    mesh = plsc.VectorSubcoreMesh(core_axis_name="c", subcore_axis_name="s")

    @functools.partial(
        pl.kernel,
        out_type=jax.ShapeDtypeStruct((NC, NP, D), jnp.float32),
        mesh=mesh,
        scratch_types=[
            pltpu.VMEM_SHARED((NP, D), jnp.float32),
            pltpu.VMEM((B,), jnp.int32),
            pltpu.VMEM((1, B), jnp.int32),
            pltpu.VMEM((B, D), jnp.float32),
            pltpu.VMEM((B, D), jnp.float32),
            pltpu.SemaphoreType.DMA,
        ],
    )
    def k(h_hbm, ea_hbm, src_hbm, dst_hbm, out_hbm,
          agg_spm, srcix, dstix, rows, eav, sem):
        c = lax.axis_index("c")
        s = lax.axis_index("s")

        def zbody(i, carry):
            for j in range(D // 16):
                eav[i, pl.ds(j * 16, 16)] = jnp.zeros((16,), jnp.float32)
            return carry
        lax.fori_loop(0, B, zbody, 0)

        def zcopy(q, carry):
            pltpu.sync_copy(eav, agg_spm.at[pl.ds(s * WPT + q * B, B)])
            return carry
        lax.fori_loop(0, WPT // B, zcopy, 0)
        plsc.subcore_barrier()

        def chunk(kk, carry):
            e0 = pl.multiple_of((c * NS + s) * EPT + kk * B, 8)
            pltpu.sync_copy(src_hbm.at[pl.ds(e0, B)], srcix)
            pltpu.sync_copy(dst_hbm.at[pl.ds(e0, B)], dstix.at[0])
            pltpu.sync_copy(ea_hbm.at[pl.ds(e0, B)], eav)
            pltpu.async_copy(h_hbm.at[srcix], rows, sem).wait()

            def cbody(bi, cc):
                for j in range(D // 16):
                    sl = pl.ds(j * 16, 16)
                    eav[bi, sl] = jnp.maximum(rows[bi, sl] + eav[bi, sl], 0.0)
                return cc
            lax.fori_loop(0, B, cbody, 0)

            pltpu.sync_copy(eav, agg_spm.at[dstix.at[0]], add=True)
            return carry
        lax.fori_loop(0, NCH, chunk, 0)
        plsc.subcore_barrier()

        def wcopy(q, carry):
            r0 = s * WPT + q * B
            pltpu.sync_copy(agg_spm.at[pl.ds(r0, B)], eav)
            pltpu.sync_copy(eav, out_hbm.at[c, pl.ds(pl.multiple_of(r0, 8),
                                                     B)])
            return carry
        lax.fori_loop(0, WPT // B, wcopy, 0)

    return k(h, edge_attr, src, dst)


def _tc_layer(h, agg0, agg1, Wl, bl, gl, betal, epsl, final):
    def body(h_ref, a0_ref, a1_ref, w_ref, b_ref, g_ref, bt_ref, e_ref,
             o_ref):
        t = (1.0 + e_ref[0, 0]) * h_ref[...] + (a0_ref[...] + a1_ref[...])
        z = jnp.dot(t, w_ref[...], preferred_element_type=jnp.float32)
        z = z + b_ref[...]
        m = jnp.mean(z, axis=0, keepdims=True)
        v = jnp.mean(jnp.square(z - m), axis=0, keepdims=True)
        zn = (z - m) * lax.rsqrt(v + 1e-5) * g_ref[...] + bt_ref[...]
        o = zn + h_ref[...]
        if final:
            o = jnp.maximum(o, 0.0)
        o_ref[...] = o

    return pl.pallas_call(
        body,
        out_shape=jax.ShapeDtypeStruct((N, D), jnp.float32),
    )(h, agg0, agg1, Wl, bl.reshape(1, D), gl.reshape(1, D),
      betal.reshape(1, D), epsl.reshape(1, 1))


def kernel(x, edge_index, edge_attr, eps, W, b, gamma, beta):
    src = edge_index[0]
    dst = edge_index[1]
    h = x
    for l in range(L):
        agg = _sc_agg(h, edge_attr, src, dst)
        h = _tc_layer(h, agg[0, :N], agg[1, :N], W[l], b[l], gamma[l],
                      beta[l], eps[l], final=(l == L - 1))
    return h


# speedup vs baseline: 7.4381x; 2.4890x over previous
import functools

import jax
import jax.numpy as jnp
from jax import lax
from jax.experimental import pallas as pl
from jax.experimental.pallas import tpu as pltpu
from jax.experimental.pallas import tpu_sc as plsc

N = 10000
E = 320000
D = 128
L = 3

NC = 2
NS = 16
NP = 10240
WPT = NP // NS
EPSC = E // NC
EPT = EPSC // NS
B = 80
NCH = EPT // B


def _sc_agg(h, edge_attr, src, dst):
    mesh = plsc.VectorSubcoreMesh(core_axis_name="c", subcore_axis_name="s")

    @functools.partial(
        pl.kernel,
        out_type=jax.ShapeDtypeStruct((NC, NP, D), jnp.float32),
        mesh=mesh,
        scratch_types=[
            pltpu.VMEM_SHARED((NP, D), jnp.float32),
            pltpu.VMEM((B,), jnp.int32),
            pltpu.VMEM((B,), jnp.int32),
            pltpu.VMEM((1, B), jnp.int32),
            pltpu.VMEM((1, B), jnp.int32),
            pltpu.VMEM((B, D), jnp.float32),
            pltpu.VMEM((B, D), jnp.float32),
            pltpu.VMEM((B, D), jnp.float32),
            pltpu.VMEM((B, D), jnp.float32),
            pltpu.SemaphoreType.DMA,
            pltpu.SemaphoreType.DMA,
            pltpu.SemaphoreType.DMA,
            pltpu.SemaphoreType.DMA,
            pltpu.SemaphoreType.DMA,
            pltpu.SemaphoreType.DMA,
            pltpu.SemaphoreType.DMA,
            pltpu.SemaphoreType.DMA,
        ],
    )
    def k(h_hbm, ea_hbm, src_hbm, dst_hbm, out_hbm, agg_spm,
          srcix0, srcix1, dstix0, dstix1, rows0, rows1, eav0, eav1,
          isem0, isem1, dsem0, dsem1, esem0, esem1, gsem0, gsem1):
        c = lax.axis_index("c")
        s = lax.axis_index("s")
        base = (c * NS + s) * EPT

        slots = ((srcix0, dstix0, rows0, eav0, isem0, dsem0, esem0, gsem0),
                 (srcix1, dstix1, rows1, eav1, isem1, dsem1, esem1, gsem1))

        def e_off(kc):
            return pl.multiple_of(base + kc * B, 8)

        def issue_srcix(kc, u):
            pltpu.async_copy(src_hbm.at[pl.ds(e_off(kc), B)], slots[u][0],
                             slots[u][4])

        def wait_srcix(u):
            pltpu.make_async_copy(src_hbm.at[pl.ds(0, B)], slots[u][0],
                                  slots[u][4]).wait()

        def issue_dstix(kc, u):
            pltpu.async_copy(dst_hbm.at[pl.ds(e_off(kc), B)],
                             slots[u][1].at[0], slots[u][5])

        def wait_dstix(u):
            pltpu.make_async_copy(dst_hbm.at[pl.ds(0, B)],
                                  slots[u][1].at[0], slots[u][5]).wait()

        def issue_ea(kc, u):
            pltpu.async_copy(ea_hbm.at[pl.ds(e_off(kc), B)], slots[u][3],
                             slots[u][6])

        def wait_ea(u):
            pltpu.make_async_copy(ea_hbm.at[pl.ds(0, B)], slots[u][3],
                                  slots[u][6]).wait()

        def issue_gather(u):
            pltpu.async_copy(h_hbm.at[slots[u][0]], slots[u][2],
                             slots[u][7])

        def wait_gather(u):
            pltpu.make_async_copy(ea_hbm.at[pl.ds(0, B)], slots[u][2],
                                  slots[u][7]).wait()

        def compute(u):
            rows, eav = slots[u][2], slots[u][3]

            @plsc.parallel_loop(0, B, unroll=2)
            def _(bi):
                for j in range(D // 16):
                    sl = pl.ds(j * 16, 16)
                    eav[bi, sl] = jnp.maximum(rows[bi, sl] + eav[bi, sl],
                                              0.0)

        def scatter(u):
            pltpu.sync_copy(slots[u][3], agg_spm.at[slots[u][1].at[0]],
                            add=True)

        def zbody(i, carry):
            for j in range(D // 16):
                rows0[i, pl.ds(j * 16, 16)] = jnp.zeros((16,), jnp.float32)
            return carry
        lax.fori_loop(0, B, zbody, 0)

        def zcopy(q, carry):
            pltpu.sync_copy(rows0, agg_spm.at[pl.ds(s * WPT + q * B, B)])
            return carry
        lax.fori_loop(0, WPT // B, zcopy, 0)

        for u in (0, 1):
            issue_srcix(u, u)
            issue_dstix(u, u)
            issue_ea(u, u)
        for u in (0, 1):
            wait_srcix(u)
            issue_gather(u)
        plsc.subcore_barrier()

        def pair(t, carry):
            for u in (0, 1):
                x = 2 * t + u
                wait_gather(u)

                @pl.when(x + 2 < NCH)
                def _():
                    issue_srcix(x + 2, u)

                wait_ea(u)
                wait_dstix(u)
                compute(u)
                scatter(u)

                @pl.when(x + 2 < NCH)
                def _():
                    issue_dstix(x + 2, u)
                    issue_ea(x + 2, u)
                    wait_srcix(u)
                    issue_gather(u)
            return carry
        lax.fori_loop(0, NCH // 2, pair, 0)

        wait_gather(0)
        wait_ea(0)
        wait_dstix(0)
        compute(0)
        scatter(0)
        plsc.subcore_barrier()

        def wcopy(q, carry):
            r0 = s * WPT + q * B
            pltpu.sync_copy(agg_spm.at[pl.ds(r0, B)], eav0)
            pltpu.sync_copy(eav0, out_hbm.at[c, pl.ds(pl.multiple_of(r0, 8),
                                                      B)])
            return carry
        lax.fori_loop(0, WPT // B, wcopy, 0)

    return k(h, edge_attr, src, dst)


def _tc_layer(h, agg0, agg1, Wl, bl, gl, betal, epsl, final):
    def body(h_ref, a0_ref, a1_ref, w_ref, b_ref, g_ref, bt_ref, e_ref,
             o_ref):
        t = (1.0 + e_ref[0, 0]) * h_ref[...] + (a0_ref[...] + a1_ref[...])
        z = jnp.dot(t, w_ref[...], preferred_element_type=jnp.float32)
        z = z + b_ref[...]
        m = jnp.mean(z, axis=0, keepdims=True)
        v = jnp.mean(jnp.square(z - m), axis=0, keepdims=True)
        zn = (z - m) * lax.rsqrt(v + 1e-5) * g_ref[...] + bt_ref[...]
        o = zn + h_ref[...]
        if final:
            o = jnp.maximum(o, 0.0)
        o_ref[...] = o

    return pl.pallas_call(
        body,
        out_shape=jax.ShapeDtypeStruct((N, D), jnp.float32),
    )(h, agg0, agg1, Wl, bl.reshape(1, D), gl.reshape(1, D),
      betal.reshape(1, D), epsl.reshape(1, 1))


def kernel(x, edge_index, edge_attr, eps, W, b, gamma, beta):
    src = edge_index[0]
    dst = edge_index[1]
    h = x
    for l in range(L):
        agg = _sc_agg(h, edge_attr, src, dst)
        h = _tc_layer(h, agg[0, :N], agg[1, :N], W[l], b[l], gamma[l],
                      beta[l], eps[l], final=(l == L - 1))
    return h
